# baseline (speedup 1.0000x reference)
import functools

import jax
import jax.numpy as jnp
from jax import lax
from jax.experimental import pallas as pl
from jax.experimental.pallas import tpu as pltpu
from jax.experimental.pallas import tpu_sc as plsc


@functools.lru_cache(maxsize=None)
def _build_sc_gather(V, S_s, S_m, D, R, cs):
    info = plsc.get_sparse_core_info()
    nc, ns = info.num_cores, info.num_subcores
    nw = nc * ns
    rpw = R // nw
    n_chunks = rpw // cs
    mesh = plsc.VectorSubcoreMesh(core_axis_name="c", subcore_axis_name="s")

    @functools.partial(
        pl.kernel,
        mesh=mesh,
        out_type=jax.ShapeDtypeStruct((R, S_m + S_s, D), jnp.float32),
        scratch_types=[
            pltpu.VMEM((2 * n_chunks, cs // 2), jnp.int32),
            pltpu.VMEM((cs, S_s, D), jnp.float32),
            pltpu.VMEM((cs, S_s, D), jnp.float32),
            pltpu.VMEM((cs, S_s, D), jnp.float32),
            pltpu.SemaphoreType.DMA,
            pltpu.SemaphoreType.DMA,
            pltpu.SemaphoreType.DMA,
            pltpu.SemaphoreType.DMA,
            pltpu.SemaphoreType.DMA,
            pltpu.SemaphoreType.DMA,
        ],
    )
    def k(ids2d_hbm, table_hbm, out_hbm,
          idx_v, g0, g1, g2, gs0, gs1, gs2, ws0, ws1, ws2):
        wid = lax.axis_index("s") * nc + lax.axis_index("c")
        base = wid * rpw
        gath_v = (g0, g1, g2)
        gsem = (gs0, gs1, gs2)
        wsem = (ws0, ws1, ws2)

        pltpu.sync_copy(
            ids2d_hbm.at[pl.ds(wid * 2 * n_chunks, 2 * n_chunks), :], idx_v)
        h = cs // 2

        def start_gather(c, b):
            return [
                pltpu.async_copy(table_hbm.at[idx_v.at[2 * c]],
                                 gath_v[b].at[pl.ds(0, h)], gsem[b]),
                pltpu.async_copy(table_hbm.at[idx_v.at[2 * c + 1]],
                                 gath_v[b].at[pl.ds(h, h)], gsem[b]),
            ]

        g = [start_gather(0, 0), start_gather(1, 1), None]
        w = [None, None, None]
        for c in range(n_chunks):
            b = c % 3
            pb = (c + 2) % 3
            if c + 2 < n_chunks:
                if w[pb] is not None:
                    w[pb].wait()
                g[pb] = start_gather(c + 2, pb)
            row0 = base + c * cs
            for d in g[b]:
                d.wait()
            w[b] = pltpu.async_copy(
                gath_v[b], out_hbm.at[pl.ds(row0, cs), pl.ds(S_m, S_s), :],
                wsem[b])
        for b in range(3):
            if w[b] is not None:
                w[b].wait()

    return k


@functools.lru_cache(maxsize=None)
def _build_tc_market_fill(S_s, S_m, D, R, mrep):
    n_dmas = R // mrep

    def body(io_in_ref, mk_ref, io_out_ref, rep_ref, sem):
        del io_in_ref
        rep_ref[0] = mk_ref[...]
        k = 1
        while k < min(mrep, 32):
            rep_ref[pl.ds(k, k)] = rep_ref[pl.ds(0, k)]
            k *= 2
        while k < mrep:
            pltpu.make_async_copy(
                rep_ref.at[pl.ds(0, k)], rep_ref.at[pl.ds(k, k)], sem
            ).start()
            pltpu.make_async_copy(
                rep_ref.at[pl.ds(0, k)], rep_ref.at[pl.ds(k, k)], sem
            ).wait()
            k *= 2
        dmas = [
            pltpu.make_async_copy(
                rep_ref,
                io_out_ref.at[pl.ds(j * mrep, mrep), pl.ds(0, S_m), :],
                sem)
            for j in range(n_dmas)
        ]
        for d in dmas:
            d.start()
        for d in dmas:
            d.wait()

    return pl.pallas_call(
        body,
        out_shape=jax.ShapeDtypeStruct((R, S_m + S_s, D), jnp.float32),
        in_specs=[
            pl.BlockSpec(memory_space=pltpu.MemorySpace.HBM),
            pl.BlockSpec(memory_space=pltpu.VMEM),
        ],
        out_specs=pl.BlockSpec(memory_space=pltpu.MemorySpace.HBM),
        scratch_shapes=[
            pltpu.VMEM((mrep, S_m, D), jnp.float32),
            pltpu.SemaphoreType.DMA,
        ],
        input_output_aliases={0: 0},
    )


def kernel(market_memory, symbol_memory, symbol_ids, batch_size, num_symbols):
    S_m, D = market_memory.shape
    V, S_s, _ = symbol_memory.shape
    b, n = symbol_ids.shape
    R = b * n
    cs = 32
    ids2d = symbol_ids.reshape(R // (cs // 2), cs // 2).astype(jnp.int32)
    gath = _build_sc_gather(V, S_s, S_m, D, R, cs)(ids2d, symbol_memory)
    out = _build_tc_market_fill(S_s, S_m, D, R, 512)(gath, market_memory)
    return out.reshape(b, n, S_m + S_s, D)


# speedup vs baseline: 1.0225x; 1.0225x over previous
import functools

import jax
import jax.numpy as jnp
from jax import lax
from jax.experimental import pallas as pl
from jax.experimental.pallas import tpu as pltpu
from jax.experimental.pallas import tpu_sc as plsc


@functools.lru_cache(maxsize=None)
def _build_sc_gather(V, S_s, S_m, D, R, cs):
    info = plsc.get_sparse_core_info()
    nc, ns = info.num_cores, info.num_subcores
    nw = nc * ns
    rpw = R // nw
    n_chunks = rpw // cs
    mesh = plsc.VectorSubcoreMesh(core_axis_name="c", subcore_axis_name="s")

    @functools.partial(
        pl.kernel,
        mesh=mesh,
        out_type=jax.ShapeDtypeStruct((R, S_m + S_s, D), jnp.float32),
        scratch_types=[
            pltpu.VMEM((n_chunks, cs), jnp.int32),
            pltpu.VMEM((cs, S_s, D), jnp.float32),
            pltpu.VMEM((cs, S_s, D), jnp.float32),
            pltpu.VMEM((cs, S_s, D), jnp.float32),
            pltpu.SemaphoreType.DMA,
            pltpu.SemaphoreType.DMA,
            pltpu.SemaphoreType.DMA,
            pltpu.SemaphoreType.DMA,
            pltpu.SemaphoreType.DMA,
            pltpu.SemaphoreType.DMA,
        ],
    )
    def k(ids2d_hbm, table_hbm, out_hbm,
          idx_v, g0, g1, g2, gs0, gs1, gs2, ws0, ws1, ws2):
        wid = lax.axis_index("s") * nc + lax.axis_index("c")
        base = wid * rpw
        gath_v = (g0, g1, g2)
        gsem = (gs0, gs1, gs2)
        wsem = (ws0, ws1, ws2)

        pltpu.sync_copy(ids2d_hbm.at[pl.ds(wid * n_chunks, n_chunks), :], idx_v)

        def start_gather(c, b):
            return pltpu.async_copy(table_hbm.at[idx_v.at[c]], gath_v[b],
                                    gsem[b])

        g = [start_gather(0, 0), start_gather(1, 1), None]
        w = [None, None, None]
        for c in range(n_chunks):
            b = c % 3
            pb = (c + 2) % 3
            if c + 2 < n_chunks:
                if w[pb] is not None:
                    w[pb].wait()
                g[pb] = start_gather(c + 2, pb)
            row0 = base + c * cs
            g[b].wait()
            w[b] = pltpu.async_copy(
                gath_v[b], out_hbm.at[pl.ds(row0, cs), pl.ds(S_m, S_s), :],
                wsem[b])
        for b in range(3):
            if w[b] is not None:
                w[b].wait()

    return k


@functools.lru_cache(maxsize=None)
def _build_tc_market_fill(S_s, S_m, D, R, mrep):
    n_dmas = R // mrep

    def body(io_in_ref, mk_ref, io_out_ref, rep_ref, sem):
        del io_in_ref
        rep_ref[0] = mk_ref[...]
        k = 1
        while k < mrep:
            rep_ref[pl.ds(k, k)] = rep_ref[pl.ds(0, k)]
            k *= 2
        dmas = [
            pltpu.make_async_copy(
                rep_ref,
                io_out_ref.at[pl.ds(j * mrep, mrep), pl.ds(0, S_m), :],
                sem)
            for j in range(n_dmas)
        ]
        for d in dmas:
            d.start()
        for d in dmas:
            d.wait()

    return pl.pallas_call(
        body,
        out_shape=jax.ShapeDtypeStruct((R, S_m + S_s, D), jnp.float32),
        in_specs=[
            pl.BlockSpec(memory_space=pltpu.MemorySpace.HBM),
            pl.BlockSpec(memory_space=pltpu.VMEM),
        ],
        out_specs=pl.BlockSpec(memory_space=pltpu.MemorySpace.HBM),
        scratch_shapes=[
            pltpu.VMEM((mrep, S_m, D), jnp.float32),
            pltpu.SemaphoreType.DMA,
        ],
        input_output_aliases={0: 0},
    )


def kernel(market_memory, symbol_memory, symbol_ids, batch_size, num_symbols):
    S_m, D = market_memory.shape
    V, S_s, _ = symbol_memory.shape
    b, n = symbol_ids.shape
    R = b * n
    cs = 32
    ids2d = symbol_ids.reshape(R // cs, cs).astype(jnp.int32)
    gath = _build_sc_gather(V, S_s, S_m, D, R, cs)(ids2d, symbol_memory)
    out = _build_tc_market_fill(S_s, S_m, D, R, 128)(gath, market_memory)
    return out.reshape(b, n, S_m + S_s, D)
